# baseline (speedup 1.0000x reference)
import jax
import jax.numpy as jnp
from jax import lax
from jax.experimental import pallas as pl
from jax.experimental.pallas import tpu as pltpu
from jax.experimental.pallas import tpu_sc as plsc

N = 10000
E = 100000
F_IN = 64
H = 64
D_E = 16
G = 32
W = 128

NC = 2
NS = 16
NW = NC * NS

CH_E = 128
NCHUNK = 25
EPAD = NW * NCHUNK * CH_E
NPAD = 10240
DUMP = NPAD - 8

CB = 512
CN = 2048

_MESH = dict(core_axis_name="c", subcore_axis_name="s")


def _sc_gather(table, idx4d):
    def body(table_hbm, idx_hbm, out_hbm, idx_v, rows_v, sem):
        cid = lax.axis_index("c")
        sid = lax.axis_index("s")
        wid = sid * NC + cid
        pltpu.sync_copy(idx_hbm.at[wid], idx_v)

        @pl.loop(0, NCHUNK)
        def _chunk(j):
            pltpu.async_copy(table_hbm.at[idx_v.at[j, 0]], rows_v, sem).wait()
            pltpu.sync_copy(
                rows_v, out_hbm.at[pl.ds((wid * NCHUNK + j) * CH_E, CH_E)])

    f = pl.kernel(
        body,
        out_type=jax.ShapeDtypeStruct((EPAD, W), jnp.float32),
        mesh=plsc.VectorSubcoreMesh(**_MESH),
        scratch_types=[
            pltpu.VMEM((NCHUNK, 1, CH_E), jnp.int32),
            pltpu.VMEM((CH_E, W), jnp.float32),
            pltpu.SemaphoreType.DMA,
        ],
    )
    return f(table, idx4d)


def _sc_scatter(msg, dst4d, zinit):
    rpt = NPAD // NS

    def body(msg_hbm, dst_hbm, z_hbm, agg_hbm, dst_v, rows_v, acc_sh, sem):
        cid = lax.axis_index("c")
        sid = lax.axis_index("s")
        wid = sid * NC + cid
        pltpu.sync_copy(z_hbm.at[pl.ds(sid * rpt, rpt)],
                        acc_sh.at[pl.ds(sid * rpt, rpt)])
        pltpu.sync_copy(dst_hbm.at[wid], dst_v)
        plsc.subcore_barrier()

        @pl.loop(0, NCHUNK)
        def _chunk(j):
            pltpu.async_copy(
                msg_hbm.at[pl.ds((wid * NCHUNK + j) * CH_E, CH_E)], rows_v,
                sem).wait()
            pltpu.sync_copy(rows_v, acc_sh.at[dst_v.at[j, 0]], add=True)

        plsc.subcore_barrier()
        pltpu.sync_copy(acc_sh.at[pl.ds(sid * rpt, rpt)],
                        agg_hbm.at[cid, pl.ds(sid * rpt, rpt)])

    f = pl.kernel(
        body,
        out_type=jax.ShapeDtypeStruct((NC, NPAD, W), jnp.float32),
        mesh=plsc.VectorSubcoreMesh(**_MESH),
        scratch_types=[
            pltpu.VMEM((NCHUNK, 1, CH_E), jnp.int32),
            pltpu.VMEM((CH_E, W), jnp.float32),
            pltpu.VMEM_SHARED((NPAD, W), jnp.float32),
            pltpu.SemaphoreType.DMA,
        ],
    )
    return f(msg, dst4d, zinit)


def _msg_body(ea_ref, xj_ref, W1_ref, b1_ref, W2r_ref, B2_ref, out_ref):
    h = jnp.maximum(
        jnp.dot(ea_ref[...], W1_ref[...], preferred_element_type=jnp.float32)
        + b1_ref[...], 0.0)
    xj = xj_ref[:, :F_IN]
    z = (h[:, :, None] * xj[:, None, :]).reshape(CB, H * H)
    msg = (jnp.dot(z, W2r_ref[...], preferred_element_type=jnp.float32)
           + jnp.dot(xj, B2_ref[...], preferred_element_type=jnp.float32))
    tail = jnp.where(
        lax.broadcasted_iota(jnp.int32, (CB, W - H), 1) == 0, 1.0, 0.0)
    out_ref[...] = jnp.concatenate([msg, tail], axis=1)


def _msg(ea, xj, W1, b1, W2r, B2):
    return pl.pallas_call(
        _msg_body,
        out_shape=jax.ShapeDtypeStruct((EPAD, W), jnp.float32),
        grid=(EPAD // CB,),
        in_specs=[
            pl.BlockSpec((CB, D_E), lambda i: (i, 0)),
            pl.BlockSpec((CB, W), lambda i: (i, 0)),
            pl.BlockSpec((D_E, H), lambda i: (0, 0)),
            pl.BlockSpec((1, H), lambda i: (0, 0)),
            pl.BlockSpec((F_IN * H, H), lambda i: (0, 0)),
            pl.BlockSpec((F_IN, H), lambda i: (0, 0)),
        ],
        out_specs=pl.BlockSpec((CB, W), lambda i: (i, 0)),
    )(ea, xj, W1, b1, W2r, B2)


def _node_body(aggp_ref, cntp_ref, xin_ref, R_ref, bc_ref, out_ref):
    agg = aggp_ref[0, :, :H] + aggp_ref[1, :, :H]
    cnt = cntp_ref[0, :, H:H + 1] + cntp_ref[1, :, H:H + 1]
    agg = agg / jnp.maximum(cnt, 1.0)
    h1 = jnp.maximum(
        agg + jnp.dot(xin_ref[:, :F_IN], R_ref[...],
                      preferred_element_type=jnp.float32) + bc_ref[...], 0.0)
    pad = jnp.concatenate(
        [cnt, jnp.zeros((CN, W - H - 1), jnp.float32)], axis=1)
    out_ref[...] = jnp.concatenate([h1, pad], axis=1)


def _node(aggp, xin, R, bc):
    return pl.pallas_call(
        _node_body,
        out_shape=jax.ShapeDtypeStruct((NPAD, W), jnp.float32),
        grid=(NPAD // CN,),
        in_specs=[
            pl.BlockSpec((NC, CN, W), lambda i: (0, i, 0)),
            pl.BlockSpec((NC, CN, W), lambda i: (0, i, 0)),
            pl.BlockSpec((CN, W), lambda i: (i, 0)),
            pl.BlockSpec((H, H), lambda i: (0, 0)),
            pl.BlockSpec((1, H), lambda i: (0, 0)),
        ],
        out_specs=pl.BlockSpec((CN, W), lambda i: (i, 0)),
    )(aggp, aggp, xin, R, bc)


def _final_body(aggp_ref, h1_ref, R_ref, bc_ref, bat_ref, L1_ref,
                bl1_ref, L2_ref, bl2_ref, out_ref, pool_acc, pcnt_acc):
    i = pl.program_id(0)
    agg = aggp_ref[0, :, :H] + aggp_ref[1, :, :H]
    cnt = h1_ref[:, H:H + 1]
    agg = agg / jnp.maximum(cnt, 1.0)
    h2 = jnp.maximum(
        agg + jnp.dot(h1_ref[:, :H], R_ref[...],
                      preferred_element_type=jnp.float32) + bc_ref[...], 0.0)
    bat = bat_ref[...]
    gid = lax.broadcasted_iota(jnp.int32, (G, CN), 0)
    M = (gid == bat).astype(jnp.float32)

    @pl.when(i == 0)
    def _init():
        pool_acc[...] = jnp.zeros_like(pool_acc)
        pcnt_acc[...] = jnp.zeros_like(pcnt_acc)

    pool_acc[...] += jnp.dot(M, h2, preferred_element_type=jnp.float32)
    pcnt_acc[...] += jnp.sum(M, axis=1, keepdims=True)

    @pl.when(i == pl.num_programs(0) - 1)
    def _readout():
        pooled = pool_acc[...] / jnp.maximum(pcnt_acc[...], 1.0)
        hid = jnp.maximum(
            jnp.dot(pooled, L1_ref[...], preferred_element_type=jnp.float32)
            + bl1_ref[...], 0.0)
        out_ref[...] = (
            jnp.dot(hid, L2_ref[...], preferred_element_type=jnp.float32)
            + bl2_ref[...])


def _final(aggp, h1, R, bc, batT, L1, bl1, L2, bl2):
    return pl.pallas_call(
        _final_body,
        out_shape=jax.ShapeDtypeStruct((G, 2), jnp.float32),
        grid=(NPAD // CN,),
        in_specs=[
            pl.BlockSpec((NC, CN, W), lambda i: (0, i, 0)),
            pl.BlockSpec((CN, W), lambda i: (i, 0)),
            pl.BlockSpec((H, H), lambda i: (0, 0)),
            pl.BlockSpec((1, H), lambda i: (0, 0)),
            pl.BlockSpec((1, CN), lambda i: (0, i)),
            pl.BlockSpec((H, H), lambda i: (0, 0)),
            pl.BlockSpec((1, H), lambda i: (0, 0)),
            pl.BlockSpec((H, 2), lambda i: (0, 0)),
            pl.BlockSpec((1, 2), lambda i: (0, 0)),
        ],
        out_specs=pl.BlockSpec((G, 2), lambda i: (0, 0)),
        scratch_shapes=[
            pltpu.VMEM((G, H), jnp.float32),
            pltpu.VMEM((G, 1), jnp.float32),
        ],
    )(aggp, h1, R, bc, batT, L1, bl1, L2, bl2)


def kernel(x, edge_index, edge_attr, batch, W1, b1, W2, b2, R1, bc1, R2, bc2,
           L1, bl1, L2, bl2):
    src = edge_index[0]
    dst = edge_index[1]

    x_pad = jnp.pad(x, ((0, NPAD - N), (0, W - F_IN)))
    ea_pad = jnp.pad(edge_attr, ((0, EPAD - E), (0, 0)))
    src4d = jnp.pad(src, (0, EPAD - E)).reshape(NW, NCHUNK, 1, CH_E)
    dst4d = jnp.pad(dst, (0, EPAD - E),
                    constant_values=DUMP).reshape(NW, NCHUNK, 1, CH_E)
    batT = jnp.pad(batch, (0, NPAD - N), constant_values=G).reshape(1, NPAD)

    W2r = W2.reshape(H, F_IN, H).reshape(F_IN * H, H)
    B2 = b2.reshape(F_IN, H)
    b1r = b1.reshape(1, H)
    bc1r = bc1.reshape(1, H)
    bc2r = bc2.reshape(1, H)
    bl1r = bl1.reshape(1, H)
    bl2r = bl2.reshape(1, 2)

    zinit = jnp.zeros((NPAD, W), jnp.float32)

    xj = _sc_gather(x_pad, src4d)
    msg1 = _msg(ea_pad, xj, W1, b1r, W2r, B2)
    agg1 = _sc_scatter(msg1, dst4d, zinit)
    h1 = _node(agg1, x_pad, R1, bc1r)

    hj = _sc_gather(h1, src4d)
    msg2 = _msg(ea_pad, hj, W1, b1r, W2r, B2)
    agg2 = _sc_scatter(msg2, dst4d, zinit)

    return _final(agg2, h1, R2, bc2r, batT, L1, bl1r, L2, bl2r)


# speedup vs baseline: 2.4787x; 2.4787x over previous
import jax
import jax.numpy as jnp
from jax import lax
from jax.experimental import pallas as pl
from jax.experimental.pallas import tpu as pltpu
from jax.experimental.pallas import tpu_sc as plsc

N = 10000
E = 100000
F_IN = 64
H = 64
D_E = 16
G = 32
W = 128

NC = 2
NS = 16
NW = NC * NS

CH_E = 128
NCHUNK = 13
EPADH = NW * NCHUNK * CH_E
EPAD = 2 * EPADH
NPAD = 10240
DUMP = NPAD - 8

CB = 1024
CN = 2048

_MESH = dict(core_axis_name="c", subcore_axis_name="s")


def _sc_gather(table, idx4d):
    def body(table_hbm, idx_hbm, out_hbm, idx_v, rows_v, sem):
        cid = lax.axis_index("c")
        sid = lax.axis_index("s")
        wid = sid * NC + cid
        pltpu.sync_copy(idx_hbm.at[wid], idx_v)

        @pl.loop(0, NCHUNK)
        def _chunk(j):
            pltpu.async_copy(table_hbm.at[idx_v.at[j, 0]], rows_v, sem).wait()
            pltpu.sync_copy(
                rows_v, out_hbm.at[pl.ds((wid * NCHUNK + j) * CH_E, CH_E)])

    f = pl.kernel(
        body,
        out_type=jax.ShapeDtypeStruct((EPADH, W), jnp.float32),
        mesh=plsc.VectorSubcoreMesh(**_MESH),
        scratch_types=[
            pltpu.VMEM((NCHUNK, 1, CH_E), jnp.int32),
            pltpu.VMEM((CH_E, W), jnp.float32),
            pltpu.SemaphoreType.DMA,
        ],
    )
    return f(table, idx4d)


def _sc_scatter(msg, dst4d, zinit):
    rpt = NPAD // NS

    def body(msg_hbm, dst_hbm, z_hbm, agg_hbm, dst_v, rows_v, acc_sh, sem):
        cid = lax.axis_index("c")
        sid = lax.axis_index("s")
        wid = sid * NC + cid
        pltpu.sync_copy(z_hbm.at[pl.ds(sid * rpt, rpt)],
                        acc_sh.at[pl.ds(sid * rpt, rpt)])
        pltpu.sync_copy(dst_hbm.at[wid], dst_v)
        plsc.subcore_barrier()

        @pl.loop(0, NCHUNK)
        def _chunk(j):
            pltpu.async_copy(
                msg_hbm.at[pl.ds((wid * NCHUNK + j) * CH_E, CH_E)], rows_v,
                sem).wait()
            pltpu.sync_copy(rows_v, acc_sh.at[dst_v.at[j, 0]], add=True)

        plsc.subcore_barrier()
        pltpu.sync_copy(acc_sh.at[pl.ds(sid * rpt, rpt)],
                        agg_hbm.at[cid, pl.ds(sid * rpt, rpt)])

    f = pl.kernel(
        body,
        out_type=jax.ShapeDtypeStruct((NC, NPAD, W), jnp.float32),
        mesh=plsc.VectorSubcoreMesh(**_MESH),
        scratch_types=[
            pltpu.VMEM((NCHUNK, 1, CH_E), jnp.int32),
            pltpu.VMEM((CH_E, W), jnp.float32),
            pltpu.VMEM_SHARED((NPAD, W), jnp.float32),
            pltpu.SemaphoreType.DMA,
        ],
    )
    return f(msg, dst4d, zinit)


def _msg_body(ea_ref, xj_ref, W1_ref, b1_ref, W2r_ref, B2_ref, exp_ref,
              out_ref):
    h = jnp.maximum(
        jnp.dot(ea_ref[...], W1_ref[...], preferred_element_type=jnp.float32)
        + b1_ref[...], 0.0)
    xj = xj_ref[:, :F_IN]
    hrep = jnp.dot(h, exp_ref[...], preferred_element_type=jnp.float32)
    z = hrep * jnp.tile(xj, (1, H))
    msg = (jnp.dot(z, W2r_ref[...], preferred_element_type=jnp.float32)
           + jnp.dot(xj, B2_ref[...], preferred_element_type=jnp.float32))
    tail = jnp.where(
        lax.broadcasted_iota(jnp.int32, (CB, W - H), 1) == 0, 1.0, 0.0)
    out_ref[...] = jnp.concatenate([msg, tail], axis=1)


def _msg(ea, xj, W1, b1, W2r, B2, expand):
    return pl.pallas_call(
        _msg_body,
        out_shape=jax.ShapeDtypeStruct((EPADH, W), jnp.float32),
        grid=(EPADH // CB,),
        in_specs=[
            pl.BlockSpec((CB, D_E), lambda i: (i, 0)),
            pl.BlockSpec((CB, W), lambda i: (i, 0)),
            pl.BlockSpec((D_E, H), lambda i: (0, 0)),
            pl.BlockSpec((1, H), lambda i: (0, 0)),
            pl.BlockSpec((F_IN * H, H), lambda i: (0, 0)),
            pl.BlockSpec((F_IN, H), lambda i: (0, 0)),
            pl.BlockSpec((H, F_IN * H), lambda i: (0, 0)),
        ],
        out_specs=pl.BlockSpec((CB, W), lambda i: (i, 0)),
    )(ea, xj, W1, b1, W2r, B2, expand)


def _node_body(aggp_ref, aggq_ref, xin_ref, R_ref, bc_ref, out_ref):
    agg = (aggp_ref[0, :, :H] + aggp_ref[1, :, :H]
           + aggq_ref[0, :, :H] + aggq_ref[1, :, :H])
    cnt = (aggp_ref[0, :, H:H + 1] + aggp_ref[1, :, H:H + 1]
           + aggq_ref[0, :, H:H + 1] + aggq_ref[1, :, H:H + 1])
    agg = agg / jnp.maximum(cnt, 1.0)
    h1 = jnp.maximum(
        agg + jnp.dot(xin_ref[:, :F_IN], R_ref[...],
                      preferred_element_type=jnp.float32) + bc_ref[...], 0.0)
    pad = jnp.concatenate(
        [cnt, jnp.zeros((CN, W - H - 1), jnp.float32)], axis=1)
    out_ref[...] = jnp.concatenate([h1, pad], axis=1)


def _node(aggp, aggq, xin, R, bc):
    return pl.pallas_call(
        _node_body,
        out_shape=jax.ShapeDtypeStruct((NPAD, W), jnp.float32),
        grid=(NPAD // CN,),
        in_specs=[
            pl.BlockSpec((NC, CN, W), lambda i: (0, i, 0)),
            pl.BlockSpec((NC, CN, W), lambda i: (0, i, 0)),
            pl.BlockSpec((CN, W), lambda i: (i, 0)),
            pl.BlockSpec((H, H), lambda i: (0, 0)),
            pl.BlockSpec((1, H), lambda i: (0, 0)),
        ],
        out_specs=pl.BlockSpec((CN, W), lambda i: (i, 0)),
    )(aggp, aggq, xin, R, bc)


def _final_body(aggp_ref, aggq_ref, h1_ref, R_ref, bc_ref, bat_ref, L1_ref,
                bl1_ref, L2_ref, bl2_ref, out_ref, pool_acc, pcnt_acc):
    i = pl.program_id(0)
    agg = (aggp_ref[0, :, :H] + aggp_ref[1, :, :H]
           + aggq_ref[0, :, :H] + aggq_ref[1, :, :H])
    cnt = h1_ref[:, H:H + 1]
    agg = agg / jnp.maximum(cnt, 1.0)
    h2 = jnp.maximum(
        agg + jnp.dot(h1_ref[:, :H], R_ref[...],
                      preferred_element_type=jnp.float32) + bc_ref[...], 0.0)
    bat = bat_ref[...]
    gid = lax.broadcasted_iota(jnp.int32, (G, CN), 0)
    M = (gid == bat).astype(jnp.float32)

    @pl.when(i == 0)
    def _init():
        pool_acc[...] = jnp.zeros_like(pool_acc)
        pcnt_acc[...] = jnp.zeros_like(pcnt_acc)

    pool_acc[...] += jnp.dot(M, h2, preferred_element_type=jnp.float32)
    pcnt_acc[...] += jnp.sum(M, axis=1, keepdims=True)

    @pl.when(i == pl.num_programs(0) - 1)
    def _readout():
        pooled = pool_acc[...] / jnp.maximum(pcnt_acc[...], 1.0)
        hid = jnp.maximum(
            jnp.dot(pooled, L1_ref[...], preferred_element_type=jnp.float32)
            + bl1_ref[...], 0.0)
        out_ref[...] = (
            jnp.dot(hid, L2_ref[...], preferred_element_type=jnp.float32)
            + bl2_ref[...])


def _final(aggp, aggq, h1, R, bc, batT, L1, bl1, L2, bl2):
    return pl.pallas_call(
        _final_body,
        out_shape=jax.ShapeDtypeStruct((G, 2), jnp.float32),
        grid=(NPAD // CN,),
        in_specs=[
            pl.BlockSpec((NC, CN, W), lambda i: (0, i, 0)),
            pl.BlockSpec((NC, CN, W), lambda i: (0, i, 0)),
            pl.BlockSpec((CN, W), lambda i: (i, 0)),
            pl.BlockSpec((H, H), lambda i: (0, 0)),
            pl.BlockSpec((1, H), lambda i: (0, 0)),
            pl.BlockSpec((1, CN), lambda i: (0, i)),
            pl.BlockSpec((H, H), lambda i: (0, 0)),
            pl.BlockSpec((1, H), lambda i: (0, 0)),
            pl.BlockSpec((H, 2), lambda i: (0, 0)),
            pl.BlockSpec((1, 2), lambda i: (0, 0)),
        ],
        out_specs=pl.BlockSpec((G, 2), lambda i: (0, 0)),
        scratch_shapes=[
            pltpu.VMEM((G, H), jnp.float32),
            pltpu.VMEM((G, 1), jnp.float32),
        ],
    )(aggp, aggq, h1, R, bc, batT, L1, bl1, L2, bl2)


def kernel(x, edge_index, edge_attr, batch, W1, b1, W2, b2, R1, bc1, R2, bc2,
           L1, bl1, L2, bl2):
    src = edge_index[0]
    dst = edge_index[1]

    x_pad = jnp.pad(x, ((0, NPAD - N), (0, W - F_IN)))
    ea_pad = jnp.pad(edge_attr, ((0, EPAD - E), (0, 0)))
    eaA, eaB = ea_pad[:EPADH], ea_pad[EPADH:]
    src_pad = jnp.pad(src, (0, EPAD - E))
    dst_pad = jnp.pad(dst, (0, EPAD - E), constant_values=DUMP)
    srcA = src_pad[:EPADH].reshape(NW, NCHUNK, 1, CH_E)
    srcB = src_pad[EPADH:].reshape(NW, NCHUNK, 1, CH_E)
    dstA = dst_pad[:EPADH].reshape(NW, NCHUNK, 1, CH_E)
    dstB = dst_pad[EPADH:].reshape(NW, NCHUNK, 1, CH_E)
    batT = jnp.pad(batch, (0, NPAD - N), constant_values=G).reshape(1, NPAD)

    W2r = W2.reshape(H, F_IN, H).reshape(F_IN * H, H)
    B2 = b2.reshape(F_IN, H)
    b1r = b1.reshape(1, H)
    bc1r = bc1.reshape(1, H)
    bc2r = bc2.reshape(1, H)
    bl1r = bl1.reshape(1, H)
    bl2r = bl2.reshape(1, 2)

    zinit = jnp.zeros((NPAD, W), jnp.float32)
    expand = jnp.repeat(jnp.eye(H, dtype=jnp.float32), H, axis=1)

    xjA = _sc_gather(x_pad, srcA)
    xjB = _sc_gather(x_pad, srcB)
    msg1A = _msg(eaA, xjA, W1, b1r, W2r, B2, expand)
    msg1B = _msg(eaB, xjB, W1, b1r, W2r, B2, expand)
    agg1A = _sc_scatter(msg1A, dstA, zinit)
    agg1B = _sc_scatter(msg1B, dstB, zinit)
    h1 = _node(agg1A, agg1B, x_pad, R1, bc1r)

    hjA = _sc_gather(h1, srcA)
    hjB = _sc_gather(h1, srcB)
    msg2A = _msg(eaA, hjA, W1, b1r, W2r, B2, expand)
    msg2B = _msg(eaB, hjB, W1, b1r, W2r, B2, expand)
    agg2A = _sc_scatter(msg2A, dstA, zinit)
    agg2B = _sc_scatter(msg2B, dstB, zinit)

    return _final(agg2A, agg2B, h1, R2, bc2r, batT, L1, bl1r, L2, bl2r)
